# baseline (speedup 1.0000x reference)
import functools

import jax
import jax.numpy as jnp
import numpy as np
from jax import lax
from jax.experimental import pallas as pl
from jax.experimental.pallas import tpu as pltpu
from jax.experimental.pallas import tpu_sc as plsc

VOCAB = 1000000
BIGRAM_DIM = 64
MODEL_DIM = 1024
BATCH = 4
SEQ = 4096
N_TOK = BATCH * SEQ

NC = 2
NS = 16
NW = NC * NS
CHUNK = N_TOK // NW
GROUPS = CHUNK // 16
IDX_ROWS = CHUNK // 128

_MULT_CUR = np.int32(36313)
_MULT_PREV = np.int32(27191)
_MOD = np.int32(VOCAB - 1)


def _sc_hash_gather(tok_hbm, table_hbm, out_hbm, ext_v, idx_v, rows_v, sem):
    wid = lax.axis_index("s") * NC + lax.axis_index("c")
    base = wid * CHUNK
    is_rowstart = (base % SEQ) == 0

    ext_v[pl.ds(0, 16)] = jnp.zeros((16,), jnp.int32)

    @pl.when(is_rowstart)
    def _():
        pltpu.sync_copy(tok_hbm.at[pl.ds(base, CHUNK)], ext_v.at[pl.ds(8, CHUNK)])

    @pl.when(jnp.logical_not(is_rowstart))
    def _():
        pltpu.sync_copy(tok_hbm.at[pl.ds(base - 8, CHUNK + 8)], ext_v)

    lane = lax.iota(jnp.int32, 16)
    for i in range(GROUPS):
        cur = ext_v[pl.ds(8 + 16 * i, 16)]
        prev = ext_v[pl.ds(7 + 16 * i, 16)]
        mixed = jnp.bitwise_xor(_MULT_CUR * cur, _MULT_PREV * prev)
        rest = lax.rem(mixed, _MOD)
        rest = jnp.where(rest < 0, rest + _MOD, rest)
        pos_in_row = (base + 16 * i + lane) % SEQ
        rest = jnp.where(pos_in_row == 0, _MOD, rest)
        idx_v[i // 8, pl.ds((i % 8) * 16, 16)] = rest

    copies = [
        pltpu.async_copy(
            table_hbm.at[idx_v.at[j]], rows_v.at[pl.ds(j * 128, 128)], sem
        )
        for j in range(IDX_ROWS)
    ]
    for c in copies:
        c.wait()
    pltpu.sync_copy(rows_v, out_hbm.at[pl.ds(base, CHUNK)])


_gather_call = functools.partial(
    pl.kernel,
    mesh=plsc.VectorSubcoreMesh(core_axis_name="c", subcore_axis_name="s"),
    out_type=jax.ShapeDtypeStruct((N_TOK, BIGRAM_DIM), jnp.bfloat16),
    scratch_types=[
        pltpu.VMEM((CHUNK + 8,), jnp.int32),
        pltpu.VMEM((IDX_ROWS, 128), jnp.int32),
        pltpu.VMEM((CHUNK, BIGRAM_DIM), jnp.bfloat16),
        pltpu.SemaphoreType.DMA,
    ],
    compiler_params=pltpu.CompilerParams(use_tc_tiling_on_sc=False),
)(_sc_hash_gather)


def _tr_body(x_ref, o_ref):
    eye = jnp.eye(BIGRAM_DIM, dtype=jnp.float32)
    o_ref[...] = lax.dot_general(
        x_ref[...], eye, (((0,), (0,)), ((), ())),
        preferred_element_type=jnp.float32,
    )


_BK = 8192
_TR_GRID = -(-VOCAB // _BK)


def _untranspose_table(table_t):
    return pl.pallas_call(
        _tr_body,
        grid=(_TR_GRID,),
        in_specs=[pl.BlockSpec((BIGRAM_DIM, _BK), lambda i: (0, i))],
        out_specs=pl.BlockSpec((_BK, BIGRAM_DIM), lambda i: (i, 0)),
        out_shape=jax.ShapeDtypeStruct((VOCAB, BIGRAM_DIM), jnp.float32),
    )(table_t)


def _mm_body(x_ref, w_ref, s_ref, o_ref):
    o_ref[...] = (
        jnp.dot(
            x_ref[...].astype(jnp.float32),
            w_ref[...],
            preferred_element_type=jnp.float32,
        )
        * s_ref[0, 0]
    )


_BM = 1024


def _projection(gathered, proj_wt, scale_arr):
    return pl.pallas_call(
        _mm_body,
        grid=(N_TOK // _BM,),
        in_specs=[
            pl.BlockSpec((_BM, BIGRAM_DIM), lambda i: (i, 0)),
            pl.BlockSpec((BIGRAM_DIM, MODEL_DIM), lambda i: (0, 0)),
            pl.BlockSpec(memory_space=pltpu.SMEM),
        ],
        out_specs=pl.BlockSpec((_BM, MODEL_DIM), lambda i: (i, 0)),
        out_shape=jax.ShapeDtypeStruct((N_TOK, MODEL_DIM), jnp.float32),
    )(gathered, proj_wt, scale_arr)


def kernel(token_ids, embed_table, proj_W, scale):
    tok = token_ids.astype(jnp.int32).reshape(N_TOK)
    gathered = _gather_call(tok, embed_table.astype(jnp.bfloat16))
    scale_arr = jnp.reshape(scale.astype(jnp.float32), (1, 1))
    out = _projection(gathered, proj_W.T, scale_arr)
    return out.reshape(BATCH, SEQ, MODEL_DIM)


# speedup vs baseline: 2.6707x; 2.6707x over previous
import functools

import jax
import jax.numpy as jnp
import numpy as np
from jax import lax
from jax.experimental import pallas as pl
from jax.experimental.pallas import tpu as pltpu
from jax.experimental.pallas import tpu_sc as plsc

VOCAB = 1000000
BIGRAM_DIM = 64
MODEL_DIM = 1024
BATCH = 4
SEQ = 4096
N_TOK = BATCH * SEQ

NC = 2
NS = 16
NW = NC * NS
CHUNK = N_TOK // NW
GROUPS = CHUNK // 16
IDX_ROWS = CHUNK // 128

PAIR_ROWS = VOCAB // 2

_MULT_CUR = np.int32(36313)
_MULT_PREV = np.int32(27191)
_MOD = np.int32(VOCAB - 1)


def _sc_hash_gather(tok_hbm, table_hbm, out_hbm, ext_v, idx2_v, par_v, rows_v,
                    half_v, sem):
    wid = lax.axis_index("s") * NC + lax.axis_index("c")
    base = wid * CHUNK
    is_rowstart = (base % SEQ) == 0

    ext_v[pl.ds(0, 16)] = jnp.zeros((16,), jnp.int32)

    @pl.when(is_rowstart)
    def _():
        pltpu.sync_copy(tok_hbm.at[pl.ds(base, CHUNK)], ext_v.at[pl.ds(8, CHUNK)])

    @pl.when(jnp.logical_not(is_rowstart))
    def _():
        pltpu.sync_copy(tok_hbm.at[pl.ds(base - 8, CHUNK + 8)], ext_v)

    lane = lax.iota(jnp.int32, 16)
    for i in range(GROUPS):
        cur = ext_v[pl.ds(8 + 16 * i, 16)]
        prev = ext_v[pl.ds(7 + 16 * i, 16)]
        mixed = jnp.bitwise_xor(_MULT_CUR * cur, _MULT_PREV * prev)
        rest = lax.rem(mixed, _MOD)
        rest = jnp.where(rest < 0, rest + _MOD, rest)
        pos_in_row = (base + 16 * i + lane) % SEQ
        rest = jnp.where(pos_in_row == 0, _MOD, rest)
        idx2_v[i // 8, pl.ds((i % 8) * 16, 16)] = ((rest >> 13) << 12) + (rest & 4095)
        par_v[pl.ds(16 * i, 16)] = ((rest >> 12) & 1) * 64

    copies = [
        pltpu.async_copy(
            table_hbm.at[idx2_v.at[j]], rows_v.at[pl.ds(j * 128, 128)], sem
        )
        for j in range(IDX_ROWS)
    ]
    for c in copies:
        c.wait()

    def pick(tg, carry):
        offs = par_v[pl.ds(tg * 16, 16)]
        for b in range(16):
            t = tg * 16 + b
            off = offs[b]
            for g in range(4):
                half_v[t, pl.ds(g * 16, 16)] = rows_v[t, pl.ds(off + g * 16, 16)]
        return carry

    lax.fori_loop(0, GROUPS, pick, 0)
    pltpu.sync_copy(half_v, out_hbm.at[pl.ds(base, CHUNK)])


_gather_call = functools.partial(
    pl.kernel,
    mesh=plsc.VectorSubcoreMesh(core_axis_name="c", subcore_axis_name="s"),
    out_type=jax.ShapeDtypeStruct((N_TOK, BIGRAM_DIM), jnp.float32),
    scratch_types=[
        pltpu.VMEM((CHUNK + 8,), jnp.int32),
        pltpu.VMEM((IDX_ROWS, 128), jnp.int32),
        pltpu.VMEM((CHUNK,), jnp.int32),
        pltpu.VMEM((CHUNK, 128), jnp.float32),
        pltpu.VMEM((CHUNK, BIGRAM_DIM), jnp.float32),
        pltpu.SemaphoreType.DMA,
    ],
    compiler_params=pltpu.CompilerParams(use_tc_tiling_on_sc=False),
)(_sc_hash_gather)


def _tr_body(x_ref, o_ref):
    eye = jnp.eye(BIGRAM_DIM, dtype=jnp.float32)
    t = lax.dot_general(
        x_ref[...], eye, (((0,), (0,)), ((), ())),
        preferred_element_type=jnp.float32,
    )
    o_ref[...] = jnp.concatenate([t[: _BK // 2, :], t[_BK // 2 :, :]], axis=1)


_BK = 8192
_TR_GRID = -(-VOCAB // _BK)
PACKED_ROWS = _TR_GRID * (_BK // 2)


def _untranspose_table(table_t):
    return pl.pallas_call(
        _tr_body,
        grid=(_TR_GRID,),
        in_specs=[pl.BlockSpec((BIGRAM_DIM, _BK), lambda i: (0, i))],
        out_specs=pl.BlockSpec((_BK // 2, 2 * BIGRAM_DIM), lambda i: (i, 0)),
        out_shape=jax.ShapeDtypeStruct((PACKED_ROWS, 2 * BIGRAM_DIM), jnp.float32),
    )(table_t)


def _mm_body(x_ref, w_ref, s_ref, o_ref):
    o_ref[...] = (
        jnp.dot(x_ref[...], w_ref[...], preferred_element_type=jnp.float32)
        * s_ref[0, 0]
    )


_BM = 1024


def _projection(gathered, proj_wt, scale_arr):
    return pl.pallas_call(
        _mm_body,
        grid=(N_TOK // _BM,),
        in_specs=[
            pl.BlockSpec((_BM, BIGRAM_DIM), lambda i: (i, 0)),
            pl.BlockSpec((BIGRAM_DIM, MODEL_DIM), lambda i: (0, 0)),
            pl.BlockSpec(memory_space=pltpu.SMEM),
        ],
        out_specs=pl.BlockSpec((_BM, MODEL_DIM), lambda i: (i, 0)),
        out_shape=jax.ShapeDtypeStruct((N_TOK, MODEL_DIM), jnp.float32),
    )(gathered, proj_wt, scale_arr)


def kernel(token_ids, embed_table, proj_W, scale):
    tok = token_ids.astype(jnp.int32).reshape(N_TOK)
    table_pairs = _untranspose_table(embed_table.T)
    gathered = _gather_call(tok, table_pairs)
    scale_arr = jnp.reshape(scale.astype(jnp.float32), (1, 1))
    out = _projection(gathered, proj_W.T, scale_arr)
    return out.reshape(BATCH, SEQ, MODEL_DIM)
